# Initial kernel scaffold; baseline (speedup 1.0000x reference)
import jax
import jax.numpy as jnp
from jax.experimental import pallas as pl


def kernel(x, edge_index, W1_l, b1, W1_r, W2_l, b2, W2_r):
    raise NotImplementedError("write your pallas kernel here")


# speedup vs baseline: 10.7039x; 10.7039x over previous
import functools

import jax
import jax.numpy as jnp
from jax import lax
from jax.experimental import pallas as pl
from jax.experimental.pallas import tpu as pltpu
from jax.experimental.pallas import tpu_sc as plsc

N = 10000
E = 320000
DF = 128
DH = 32
NC = 2
NS = 16
NW = NC * NS
CH = 80
RPW = E // NW // CH
NROWS = E // CH
ZPT = N // NS

_MESH = dict(core_axis_name="c", subcore_axis_name="s", num_cores=NC,
             num_subcores=NS)
_SC_PARAMS = pltpu.CompilerParams(use_tc_tiling_on_sc=False)


def _sc_agg32(y1, src2, dst2, z32, z1, acc_out, deg_out,
              src_v, dst_v, rows_v, ones_v, acc_sh, deg_sh, gsem):
    c = lax.axis_index("c")
    s = lax.axis_index("s")
    wid = c * NS + s
    pltpu.sync_copy(z32.at[pl.ds(s * ZPT, ZPT)], acc_sh.at[pl.ds(s * ZPT, ZPT)])

    @pl.when(s < 10)
    def _():
        pltpu.sync_copy(z1.at[pl.ds(s * 1000, 1000)],
                        deg_sh.at[pl.ds(s * 1000, 1000)])

    for i in range(CH // 16):
        ones_v[pl.ds(i * 16, 16)] = jnp.full((16,), 1.0, jnp.float32)
    pltpu.sync_copy(src2.at[pl.ds(wid * RPW, RPW)], src_v)
    pltpu.sync_copy(dst2.at[pl.ds(wid * RPW, RPW)], dst_v)
    plsc.subcore_barrier()

    def body(j, carry):
        pltpu.async_copy(y1.at[src_v.at[j]], rows_v, gsem).wait()
        pltpu.sync_copy(rows_v, acc_sh.at[dst_v.at[j]], add=True)
        pltpu.sync_copy(ones_v, deg_sh.at[dst_v.at[j]], add=True)
        return carry

    lax.fori_loop(0, RPW, body, 0)
    plsc.subcore_barrier()

    @pl.when(s == 0)
    def _():
        pltpu.sync_copy(acc_sh, acc_out.at[c])
        pltpu.sync_copy(deg_sh, deg_out.at[c])


def _sc_agg1(y2, src2, dst2, z1, acc_out, src_v, dst_v, vals_v, acc_sh, gsem):
    c = lax.axis_index("c")
    s = lax.axis_index("s")
    wid = c * NS + s

    @pl.when(s < 10)
    def _():
        pltpu.sync_copy(z1.at[pl.ds(s * 1000, 1000)],
                        acc_sh.at[pl.ds(s * 1000, 1000)])

    pltpu.sync_copy(src2.at[pl.ds(wid * RPW, RPW)], src_v)
    pltpu.sync_copy(dst2.at[pl.ds(wid * RPW, RPW)], dst_v)
    plsc.subcore_barrier()

    def body(j, carry):
        pltpu.async_copy(y2.at[src_v.at[j]], vals_v, gsem).wait()
        pltpu.sync_copy(vals_v, acc_sh.at[dst_v.at[j]], add=True)
        return carry

    lax.fori_loop(0, RPW, body, 0)
    plsc.subcore_barrier()

    @pl.when(s == 0)
    def _():
        pltpu.sync_copy(acc_sh, acc_out.at[c])


def _tc_pre(x_ref, wlt_ref, wrt_ref, b1_ref, y1_ref, r1_ref):
    xv = x_ref[...]
    y1_ref[...] = jnp.dot(xv, wlt_ref[...], preferred_element_type=jnp.float32)
    r1_ref[...] = (jnp.dot(xv, wrt_ref[...], preferred_element_type=jnp.float32)
                   + b1_ref[...])


def _tc_mid(accp_ref, degp_ref, r1_ref, w2lt_ref, w2rt_ref, b2_ref,
            y2_ref, r2_ref):
    inv = 1.0 / jnp.maximum(degp_ref[0] + degp_ref[1], 1.0)
    h = jnp.maximum((accp_ref[0] + accp_ref[1]) * inv + r1_ref[...], 0.0)
    y2_ref[...] = jnp.dot(h, w2lt_ref[...], preferred_element_type=jnp.float32)
    r2_ref[...] = (jnp.dot(h, w2rt_ref[...], preferred_element_type=jnp.float32)
                   + b2_ref[...])


def _tc_post(accp_ref, degp_ref, r2_ref, out_ref):
    inv = 1.0 / jnp.maximum(degp_ref[0] + degp_ref[1], 1.0)
    out_ref[...] = (accp_ref[0] + accp_ref[1]) * inv + r2_ref[...]


def kernel(x, edge_index, W1_l, b1, W1_r, W2_l, b2, W2_r):
    ei = edge_index.astype(jnp.int32)
    src2 = ei[0].reshape(NROWS, CH)
    dst2 = ei[1].reshape(NROWS, CH)
    z32 = jnp.zeros((N, DH), jnp.float32)
    z1 = jnp.zeros((N,), jnp.float32)

    y1, r1 = pl.pallas_call(
        _tc_pre,
        out_shape=[jax.ShapeDtypeStruct((N, DH), jnp.float32),
                   jax.ShapeDtypeStruct((N, DH), jnp.float32)],
    )(x, W1_l.T, W1_r.T, b1.reshape(1, DH))

    mesh = plsc.VectorSubcoreMesh(**_MESH)
    acc1p, degp = pl.kernel(
        _sc_agg32,
        out_type=[jax.ShapeDtypeStruct((NC, N, DH), jnp.float32),
                  jax.ShapeDtypeStruct((NC, N), jnp.float32)],
        mesh=mesh,
        scratch_types=[
            pltpu.VMEM((RPW, CH), jnp.int32),
            pltpu.VMEM((RPW, CH), jnp.int32),
            pltpu.VMEM((CH, DH), jnp.float32),
            pltpu.VMEM((CH,), jnp.float32),
            pltpu.VMEM_SHARED((N, DH), jnp.float32),
            pltpu.VMEM_SHARED((N,), jnp.float32),
            pltpu.SemaphoreType.DMA,
        ],
        compiler_params=_SC_PARAMS,
    )(y1, src2, dst2, z32, z1)

    degp2 = degp.reshape(NC, N, 1)
    y2, r2 = pl.pallas_call(
        _tc_mid,
        out_shape=[jax.ShapeDtypeStruct((N, 1), jnp.float32),
                   jax.ShapeDtypeStruct((N, 1), jnp.float32)],
    )(acc1p, degp2, r1, W2_l.T, W2_r.T, b2.reshape(1, 1))

    acc2p = pl.kernel(
        _sc_agg1,
        out_type=jax.ShapeDtypeStruct((NC, N), jnp.float32),
        mesh=plsc.VectorSubcoreMesh(**_MESH),
        scratch_types=[
            pltpu.VMEM((RPW, CH), jnp.int32),
            pltpu.VMEM((RPW, CH), jnp.int32),
            pltpu.VMEM((CH,), jnp.float32),
            pltpu.VMEM_SHARED((N,), jnp.float32),
            pltpu.SemaphoreType.DMA,
        ],
        compiler_params=_SC_PARAMS,
    )(y2.reshape(N), src2, dst2, z1)

    out = pl.pallas_call(
        _tc_post,
        out_shape=jax.ShapeDtypeStruct((N, 1), jnp.float32),
    )(acc2p.reshape(NC, N, 1), degp2, r2)
    return out


# speedup vs baseline: 16.7691x; 1.5666x over previous
import functools

import jax
import jax.numpy as jnp
from jax import lax
from jax.experimental import pallas as pl
from jax.experimental.pallas import tpu as pltpu
from jax.experimental.pallas import tpu_sc as plsc

N = 10000
E = 320000
DF = 128
DH = 32
DT = 40
NC = 2
NS = 16
NW = NC * NS
CH = 80
RPW = E // NW // CH
NROWS = E // CH
NB = 5
LEAD = 2

_MESH = dict(core_axis_name="c", subcore_axis_name="s", num_cores=NC,
             num_subcores=NS)
_SC_PARAMS = pltpu.CompilerParams(use_tc_tiling_on_sc=False)


def _pipelined_agg(table, src_v, dst_v, rows_v, acc_sh, gsems, ssems):
    def _gather(j, b):
        return pltpu.async_copy(table.at[src_v.at[j]], rows_v.at[b], gsems[b])

    def _scatter(j, b):
        return pltpu.async_copy(rows_v.at[b], acc_sh.at[dst_v.at[j]],
                                ssems[b], add=True)

    for b in range(LEAD):
        _gather(b, b)

    def outer(t, carry):
        for b in range(NB):
            j = t * NB + b
            pltpu.make_async_copy(table.at[src_v.at[j]], rows_v.at[b],
                                  gsems[b]).wait()
            _scatter(j, b)
            bn = (b + LEAD) % NB
            jn = j + LEAD

            @pl.when(jnp.logical_and(jn < RPW, j >= NB - LEAD))
            def _():
                pltpu.make_async_copy(rows_v.at[bn], acc_sh.at[dst_v.at[0]],
                                      ssems[bn]).wait()

            @pl.when(jn < RPW)
            def _():
                _gather(jn, bn)
        return carry

    lax.fori_loop(0, RPW // NB, outer, 0)
    for b in range(NB):
        pltpu.make_async_copy(rows_v.at[b], acc_sh.at[dst_v.at[0]],
                              ssems[b]).wait()


def _sc_agg40(y1t, src2, dst2, z40, acc_out,
              src_v, dst_v, rows_v, acc_sh,
              g0, g1, g2, g3, g4, s0, s1, s2, s3, s4):
    c = lax.axis_index("c")
    s = lax.axis_index("s")
    wid = c * NS + s
    @pl.when(s < 10)
    def _():
        pltpu.sync_copy(z40.at[pl.ds(s * 1000, 1000)],
                        acc_sh.at[pl.ds(s * 1000, 1000)])

    pltpu.sync_copy(src2.at[pl.ds(wid * RPW, RPW)], src_v)
    pltpu.sync_copy(dst2.at[pl.ds(wid * RPW, RPW)], dst_v)
    plsc.subcore_barrier()
    _pipelined_agg(y1t, src_v, dst_v, rows_v, acc_sh,
                   [g0, g1, g2, g3, g4], [s0, s1, s2, s3, s4])
    plsc.subcore_barrier()

    @pl.when(s == 0)
    def _():
        pltpu.sync_copy(acc_sh, acc_out.at[c])


def _sc_agg1(y2, src2, dst2, z1, acc_out,
             src_v, dst_v, vals_v, acc_sh,
             g0, g1, g2, g3, g4, s0, s1, s2, s3, s4):
    c = lax.axis_index("c")
    s = lax.axis_index("s")
    wid = c * NS + s

    @pl.when(s < 10)
    def _():
        pltpu.sync_copy(z1.at[pl.ds(s * 1000, 1000)],
                        acc_sh.at[pl.ds(s * 1000, 1000)])

    pltpu.sync_copy(src2.at[pl.ds(wid * RPW, RPW)], src_v)
    pltpu.sync_copy(dst2.at[pl.ds(wid * RPW, RPW)], dst_v)
    plsc.subcore_barrier()
    _pipelined_agg(y2, src_v, dst_v, vals_v, acc_sh,
                   [g0, g1, g2, g3, g4], [s0, s1, s2, s3, s4])
    plsc.subcore_barrier()

    @pl.when(s == 0)
    def _():
        pltpu.sync_copy(acc_sh, acc_out.at[c])


def _tc_pre(xa_ref, wlt_ref, wrt_ref, y1t_ref, r1_ref):
    xv = xa_ref[...]
    y1t_ref[...] = jnp.dot(xv, wlt_ref[...],
                           preferred_element_type=jnp.float32)
    r1_ref[...] = jnp.dot(xv, wrt_ref[...],
                          preferred_element_type=jnp.float32)


def _tc_mid(accp_ref, r1_ref, w2lt_ref, w2rt_ref, b2_ref,
            y2_ref, r2_ref, invd_ref):
    sum40 = accp_ref[0] + accp_ref[1]
    inv = 1.0 / jnp.maximum(sum40[:, DH:DH + 1], 1.0)
    h = jnp.maximum(sum40[:, :DH] * inv + r1_ref[...], 0.0)
    y2_ref[...] = jnp.dot(h, w2lt_ref[...], preferred_element_type=jnp.float32)
    r2_ref[...] = (jnp.dot(h, w2rt_ref[...], preferred_element_type=jnp.float32)
                   + b2_ref[...])
    invd_ref[...] = inv


def _tc_post(accp_ref, invd_ref, r2_ref, out_ref):
    out_ref[...] = (accp_ref[0] + accp_ref[1]) * invd_ref[...] + r2_ref[...]


def kernel(x, edge_index, W1_l, b1, W1_r, W2_l, b2, W2_r):
    ei = edge_index.astype(jnp.int32)
    src2 = ei[0].reshape(NROWS, CH)
    dst2 = ei[1].reshape(NROWS, CH)
    z40 = jnp.zeros((N, DT), jnp.float32)
    z1 = jnp.zeros((N,), jnp.float32)

    xa = jnp.concatenate([x, jnp.ones((N, 1), jnp.float32)], axis=1)
    wlt = jnp.zeros((DF + 1, DT), jnp.float32)
    wlt = wlt.at[:DF, :DH].set(W1_l.T).at[DF, DH].set(1.0)
    wrt = jnp.concatenate([W1_r.T, b1.reshape(1, DH)], axis=0)

    y1t, r1 = pl.pallas_call(
        _tc_pre,
        out_shape=[jax.ShapeDtypeStruct((N, DT), jnp.float32),
                   jax.ShapeDtypeStruct((N, DH), jnp.float32)],
    )(xa, wlt, wrt)

    sc1_scratch = [
        pltpu.VMEM((RPW, CH), jnp.int32),
        pltpu.VMEM((RPW, CH), jnp.int32),
        pltpu.VMEM((NB, CH, DT), jnp.float32),
        pltpu.VMEM_SHARED((N, DT), jnp.float32),
    ] + [pltpu.SemaphoreType.DMA] * (2 * NB)
    acc1p = pl.kernel(
        _sc_agg40,
        out_type=jax.ShapeDtypeStruct((NC, N, DT), jnp.float32),
        mesh=plsc.VectorSubcoreMesh(**_MESH),
        scratch_types=sc1_scratch,
        compiler_params=_SC_PARAMS,
    )(y1t, src2, dst2, z40)

    y2, r2, invd = pl.pallas_call(
        _tc_mid,
        out_shape=[jax.ShapeDtypeStruct((N, 1), jnp.float32),
                   jax.ShapeDtypeStruct((N, 1), jnp.float32),
                   jax.ShapeDtypeStruct((N, 1), jnp.float32)],
    )(acc1p, r1, W2_l.T, W2_r.T, b2.reshape(1, 1))

    sc2_scratch = [
        pltpu.VMEM((RPW, CH), jnp.int32),
        pltpu.VMEM((RPW, CH), jnp.int32),
        pltpu.VMEM((NB, CH), jnp.float32),
        pltpu.VMEM_SHARED((N,), jnp.float32),
    ] + [pltpu.SemaphoreType.DMA] * (2 * NB)
    acc2p = pl.kernel(
        _sc_agg1,
        out_type=jax.ShapeDtypeStruct((NC, N), jnp.float32),
        mesh=plsc.VectorSubcoreMesh(**_MESH),
        scratch_types=sc2_scratch,
        compiler_params=_SC_PARAMS,
    )(y2.reshape(N), src2, dst2, z1)

    out = pl.pallas_call(
        _tc_post,
        out_shape=jax.ShapeDtypeStruct((N, 1), jnp.float32),
    )(acc2p.reshape(NC, N, 1), invd, r2)
    return out
